# Initial kernel scaffold; baseline (speedup 1.0000x reference)
import jax
import jax.numpy as jnp
from jax.experimental import pallas as pl


def kernel(feat, segment_ids):
    raise NotImplementedError("write your pallas kernel here")


# speedup vs baseline: 3.0716x; 3.0716x over previous
import functools

import jax
import jax.numpy as jnp
from jax import lax
from jax.experimental import pallas as pl
from jax.experimental.pallas import tpu as pltpu
from jax.experimental.pallas import tpu_sc as plsc

EPS = 1e-5
N = 100000
D = 256
S = 64
NC = 2
NS = 16
NW = NC * NS
B = 128
NG = D // 16

_HEX_Q, _HEX_R = divmod(N // 16, NW)
IDS_CHUNK = 16 * (_HEX_Q + 1)
IDS_PAD = 16 * _HEX_Q * NW + IDS_CHUNK
N_CHUNKS = (IDS_CHUNK + B - 1) // B


def _sc_partials(feat, ids):
  mesh = plsc.VectorSubcoreMesh(core_axis_name="c", subcore_axis_name="s")

  @functools.partial(
      pl.kernel,
      mesh=mesh,
      out_type=[
          jax.ShapeDtypeStruct((NW, S, D), jnp.float32),
          jax.ShapeDtypeStruct((NW, S, D), jnp.float32),
      ],
      scratch_types=[
          pltpu.VMEM((IDS_CHUNK,), jnp.int32),
          pltpu.VMEM((B, D), jnp.float32),
          pltpu.VMEM((S, D), jnp.float32),
          pltpu.VMEM((S, D), jnp.float32),
      ],
  )
  def k(feat_hbm, ids_hbm, out_sum, out_sq, ids_v, buf, acc_sum, acc_sq):
    cid = lax.axis_index("c")
    sid = lax.axis_index("s")
    wid = sid * NC + cid
    lo = 16 * (_HEX_Q * wid + jnp.minimum(wid, _HEX_R))
    rpw = 16 * (_HEX_Q + jnp.where(wid < _HEX_R, 1, 0))
    hi = lo + rpw

    pltpu.sync_copy(ids_hbm.at[pl.ds(lo, IDS_CHUNK)], ids_v)

    def zero_row(i, carry):
      z = jnp.zeros((16,), jnp.float32)
      for g in range(NG):
        acc_sum[i, pl.ds(g * 16, 16)] = z
        acc_sq[i, pl.ds(g * 16, 16)] = z
      return carry

    lax.fori_loop(0, S, zero_row, 0)

    def chunk_body(ci, carry):
      start = lo + ci * B
      end = jnp.minimum(start + B, hi)
      base = end - B
      pltpu.sync_copy(feat_hbm.at[pl.ds(base, B), :], buf)
      nsb = (end - start) // 16

      def sub_body(sb, c2):
        row0 = start + sb * 16
        segv = ids_v[pl.ds(row0 - lo, 16)]
        segs = [segv[j] for j in range(16)]
        rl0 = row0 - base

        def g_body(g, c3):
          g16 = g * 16
          for j in range(16):
            v = buf[rl0 + j, pl.ds(g16, 16)]
            plsc.addupdate(acc_sum.at[segs[j], pl.ds(g16, 16)], v)
            plsc.addupdate(acc_sq.at[segs[j], pl.ds(g16, 16)], v * v)
          return c3

        lax.fori_loop(0, NG, g_body, 0)
        return c2

      lax.fori_loop(0, nsb, sub_body, 0)
      return carry

    lax.fori_loop(0, N_CHUNKS, chunk_body, 0)

    pltpu.sync_copy(acc_sum, out_sum.at[wid])
    pltpu.sync_copy(acc_sq, out_sq.at[wid])

  return k(feat, ids)


def _finalize_body(sum_ref, sq_ref, out_ref):
  s = jnp.sum(sum_ref[...], axis=0)
  q = jnp.sum(sq_ref[...], axis=0)
  out_ref[...] = jnp.sqrt(jax.nn.relu(q - s * s) + EPS)


@jax.jit
def kernel(feat, segment_ids):
  ids = segment_ids.astype(jnp.int32)
  ids = jnp.pad(ids, (0, IDS_PAD - N), constant_values=0)
  part_sum, part_sq = _sc_partials(feat, ids)
  return pl.pallas_call(
      _finalize_body,
      out_shape=jax.ShapeDtypeStruct((S, D), jnp.float32),
  )(part_sum, part_sq)


# speedup vs baseline: 6.1779x; 2.0113x over previous
import functools

import jax
import jax.numpy as jnp
from jax import lax
from jax.experimental import pallas as pl
from jax.experimental.pallas import tpu as pltpu
from jax.experimental.pallas import tpu_sc as plsc

EPS = 1e-5
N = 100000
D = 256
S = 64
NC = 2
NS = 16
NW = NC * NS
B = 176
NG = D // 16

_HEX_Q, _HEX_R = divmod(N // 16, NW)
IDS_CHUNK = 16 * (_HEX_Q + 1)
IDS_PAD = 16 * _HEX_Q * NW + IDS_CHUNK
N_CHUNKS = -(-IDS_CHUNK // B)


def _sc_partials(feat, ids):
  mesh = plsc.VectorSubcoreMesh(core_axis_name="c", subcore_axis_name="s")

  @functools.partial(
      pl.kernel,
      mesh=mesh,
      out_type=[
          jax.ShapeDtypeStruct((NW, S, D), jnp.float32),
          jax.ShapeDtypeStruct((NW, S, D), jnp.float32),
      ],
      scratch_types=[
          pltpu.VMEM((IDS_CHUNK,), jnp.int32),
          pltpu.VMEM((B, D), jnp.float32),
          pltpu.VMEM((S, D), jnp.float32),
          pltpu.VMEM((S, D), jnp.float32),
          pltpu.VMEM((D,), jnp.float32),
          pltpu.VMEM((D,), jnp.float32),
      ],
  )
  def k(feat_hbm, ids_hbm, out_sum, out_sq, ids_v, buf, acc_sum, acc_sq,
        car_sum, car_sq):
    cid = lax.axis_index("c")
    sid = lax.axis_index("s")
    wid = sid * NC + cid
    lo = 16 * (_HEX_Q * wid + jnp.minimum(wid, _HEX_R))
    rpw = 16 * (_HEX_Q + jnp.where(wid < _HEX_R, 1, 0))
    hi = lo + rpw

    pltpu.sync_copy(ids_hbm.at[pl.ds(lo, IDS_CHUNK)], ids_v)

    z = jnp.zeros((16,), jnp.float32)
    for g in range(NG):
      car_sum[pl.ds(g * 16, 16)] = z
      car_sq[pl.ds(g * 16, 16)] = z

    def zero_row(i, carry):
      zz = jnp.zeros((16,), jnp.float32)
      for g in range(NG):
        acc_sum[i, pl.ds(g * 16, 16)] = zz
        acc_sq[i, pl.ds(g * 16, 16)] = zz
      return carry

    lax.fori_loop(0, S, zero_row, 0)

    cur0 = ids_v[pl.ds(0, 16)][0]

    def chunk_body(ci, cur):
      start = lo + ci * B
      end = jnp.minimum(start + B, hi)
      base = end - B
      pltpu.sync_copy(feat_hbm.at[pl.ds(base, B), :], buf)
      nsb = (end - start) // 16

      def sub_body(sb, cur):
        row0 = start + sb * 16
        segv = ids_v[pl.ds(row0 - lo, 16)]
        first = segv[0]
        last = segv[15]
        rl0 = row0 - base
        is_fast = jnp.logical_and(first == cur, first == last)
        n_slow = jnp.where(is_fast, 0, 1)

        def slow_body(_, c2):
          for g in range(NG):
            plsc.addupdate(acc_sum.at[cur, pl.ds(g * 16, 16)],
                           car_sum[pl.ds(g * 16, 16)])
            plsc.addupdate(acc_sq.at[cur, pl.ds(g * 16, 16)],
                           car_sq[pl.ds(g * 16, 16)])
            car_sum[pl.ds(g * 16, 16)] = jnp.zeros((16,), jnp.float32)
            car_sq[pl.ds(g * 16, 16)] = jnp.zeros((16,), jnp.float32)
          sv = ids_v[pl.ds(row0 - lo, 16)]
          for j in range(16):
            seg = sv[j]
            for g in range(NG):
              v = buf[rl0 + j, pl.ds(g * 16, 16)]
              plsc.addupdate(acc_sum.at[seg, pl.ds(g * 16, 16)], v)
              plsc.addupdate(acc_sq.at[seg, pl.ds(g * 16, 16)], v * v)
          return c2

        lax.fori_loop(0, n_slow, slow_body, 0)

        def fast_body(_, c2):
          for g in range(NG):
            s_acc = car_sum[pl.ds(g * 16, 16)]
            q_acc = car_sq[pl.ds(g * 16, 16)]
            for j in range(16):
              v = buf[rl0 + j, pl.ds(g * 16, 16)]
              s_acc = s_acc + v
              q_acc = q_acc + v * v
            car_sum[pl.ds(g * 16, 16)] = s_acc
            car_sq[pl.ds(g * 16, 16)] = q_acc
          return c2

        lax.fori_loop(0, 1 - n_slow, fast_body, 0)

        return last

      return lax.fori_loop(0, nsb, sub_body, cur)

    cur = lax.fori_loop(0, N_CHUNKS, chunk_body, cur0)

    for g in range(NG):
      plsc.addupdate(acc_sum.at[cur, pl.ds(g * 16, 16)],
                     car_sum[pl.ds(g * 16, 16)])
      plsc.addupdate(acc_sq.at[cur, pl.ds(g * 16, 16)],
                     car_sq[pl.ds(g * 16, 16)])

    pltpu.sync_copy(acc_sum, out_sum.at[wid])
    pltpu.sync_copy(acc_sq, out_sq.at[wid])

  return k(feat, ids)


def _finalize_body(sum_ref, sq_ref, out_ref):
  s = jnp.sum(sum_ref[...], axis=0)
  q = jnp.sum(sq_ref[...], axis=0)
  out_ref[...] = jnp.sqrt(jax.nn.relu(q - s * s) + EPS)


@jax.jit
def kernel(feat, segment_ids):
  ids = segment_ids.astype(jnp.int32)
  ids = jnp.pad(ids, (0, IDS_PAD - N), constant_values=0)
  part_sum, part_sq = _sc_partials(feat, ids)
  return pl.pallas_call(
      _finalize_body,
      out_shape=jax.ShapeDtypeStruct((S, D), jnp.float32),
  )(part_sum, part_sq)


# speedup vs baseline: 8.3476x; 1.3512x over previous
import functools

import jax
import jax.numpy as jnp
from jax import lax
from jax.experimental import pallas as pl
from jax.experimental.pallas import tpu as pltpu
from jax.experimental.pallas import tpu_sc as plsc

EPS = 1e-5
N = 100000
D = 256
S = 64
NC = 2
NS = 16
NW = NC * NS
B = 176
NG = D // 16

_HEX_Q, _HEX_R = divmod(N // 16, NW)
IDS_CHUNK = 16 * (_HEX_Q + 1)
IDS_PAD = 16 * _HEX_Q * NW + IDS_CHUNK
N_CHUNKS = -(-IDS_CHUNK // B)


def _sc_partials(feat, ids):
  mesh = plsc.VectorSubcoreMesh(core_axis_name="c", subcore_axis_name="s")

  @functools.partial(
      pl.kernel,
      mesh=mesh,
      out_type=[
          jax.ShapeDtypeStruct((NW, S, D), jnp.float32),
          jax.ShapeDtypeStruct((NW, S, D), jnp.float32),
      ],
      scratch_types=[
          pltpu.VMEM((IDS_CHUNK,), jnp.int32),
          pltpu.VMEM((2, B, D), jnp.float32),
          pltpu.VMEM((S, D), jnp.float32),
          pltpu.VMEM((S, D), jnp.float32),
          pltpu.VMEM((D,), jnp.float32),
          pltpu.VMEM((D,), jnp.float32),
          pltpu.SemaphoreType.DMA((2,)),
      ],
  )
  def k(feat_hbm, ids_hbm, out_sum, out_sq, ids_v, buf, acc_sum, acc_sq,
        car_sum, car_sq, sems):
    cid = lax.axis_index("c")
    sid = lax.axis_index("s")
    wid = sid * NC + cid
    lo = 16 * (_HEX_Q * wid + jnp.minimum(wid, _HEX_R))
    rpw = 16 * (_HEX_Q + jnp.where(wid < _HEX_R, 1, 0))
    hi = lo + rpw

    pltpu.sync_copy(ids_hbm.at[pl.ds(lo, IDS_CHUNK)], ids_v)

    z = jnp.zeros((16,), jnp.float32)
    for g in range(NG):
      car_sum[pl.ds(g * 16, 16)] = z
      car_sq[pl.ds(g * 16, 16)] = z

    def zero_row(i, carry):
      zz = jnp.zeros((16,), jnp.float32)
      for g in range(NG):
        acc_sum[i, pl.ds(g * 16, 16)] = zz
        acc_sq[i, pl.ds(g * 16, 16)] = zz
      return carry

    lax.fori_loop(0, S, zero_row, 0)

    cur0 = ids_v[pl.ds(0, 16)][0]

    def chunk_base(ci):
      return jnp.minimum(lo + ci * B + B, hi) - B

    def start_copy(ci, par):
      pltpu.async_copy(feat_hbm.at[pl.ds(chunk_base(ci), B), :], buf.at[par],
                       sems.at[par])

    start_copy(0, 0)

    def chunk_body(ci, cur):
      start = lo + ci * B
      end = jnp.minimum(start + B, hi)
      base = end - B
      par = lax.rem(ci, 2)
      pltpu.make_async_copy(feat_hbm.at[pl.ds(base, B), :], buf.at[par],
                            sems.at[par]).wait()
      nxt = jnp.minimum(ci + 1, N_CHUNKS - 1)
      start_copy(nxt, 1 - par)
      nsb = (end - start) // 16

      def sub_body(sb, cur):
        row0 = start + sb * 16
        segv = ids_v[pl.ds(row0 - lo, 16)]
        first = segv[0]
        last = segv[15]
        rl0 = row0 - base
        is_fast = jnp.logical_and(first == cur, first == last)
        n_slow = jnp.where(is_fast, 0, 1)

        def slow_body(_, c2):
          for g in range(NG):
            plsc.addupdate(acc_sum.at[cur, pl.ds(g * 16, 16)],
                           car_sum[pl.ds(g * 16, 16)])
            plsc.addupdate(acc_sq.at[cur, pl.ds(g * 16, 16)],
                           car_sq[pl.ds(g * 16, 16)])
            car_sum[pl.ds(g * 16, 16)] = jnp.zeros((16,), jnp.float32)
            car_sq[pl.ds(g * 16, 16)] = jnp.zeros((16,), jnp.float32)
          sv = ids_v[pl.ds(row0 - lo, 16)]
          for j in range(16):
            seg = sv[j]
            for g in range(NG):
              v = buf[par, rl0 + j, pl.ds(g * 16, 16)]
              plsc.addupdate(acc_sum.at[seg, pl.ds(g * 16, 16)], v)
              plsc.addupdate(acc_sq.at[seg, pl.ds(g * 16, 16)], v * v)
          return c2

        lax.fori_loop(0, n_slow, slow_body, 0)

        def fast_body(_, c2):
          for g in range(NG):
            s_acc = car_sum[pl.ds(g * 16, 16)]
            q_acc = car_sq[pl.ds(g * 16, 16)]
            for j in range(16):
              v = buf[par, rl0 + j, pl.ds(g * 16, 16)]
              s_acc = s_acc + v
              q_acc = q_acc + v * v
            car_sum[pl.ds(g * 16, 16)] = s_acc
            car_sq[pl.ds(g * 16, 16)] = q_acc
          return c2

        lax.fori_loop(0, 1 - n_slow, fast_body, 0)

        return last

      return lax.fori_loop(0, nsb, sub_body, cur)

    cur = lax.fori_loop(0, N_CHUNKS, chunk_body, cur0)

    pltpu.make_async_copy(
        feat_hbm.at[pl.ds(chunk_base(N_CHUNKS - 1), B), :], buf.at[0],
        sems.at[0]).wait()

    for g in range(NG):
      plsc.addupdate(acc_sum.at[cur, pl.ds(g * 16, 16)],
                     car_sum[pl.ds(g * 16, 16)])
      plsc.addupdate(acc_sq.at[cur, pl.ds(g * 16, 16)],
                     car_sq[pl.ds(g * 16, 16)])

    pltpu.sync_copy(acc_sum, out_sum.at[wid])
    pltpu.sync_copy(acc_sq, out_sq.at[wid])

  return k(feat, ids)


def _finalize_body(sum_ref, sq_ref, out_ref):
  s = jnp.sum(sum_ref[...], axis=0)
  q = jnp.sum(sq_ref[...], axis=0)
  out_ref[...] = jnp.sqrt(jax.nn.relu(q - s * s) + EPS)


@jax.jit
def kernel(feat, segment_ids):
  ids = segment_ids.astype(jnp.int32)
  ids = jnp.pad(ids, (0, IDS_PAD - N), constant_values=0)
  part_sum, part_sq = _sc_partials(feat, ids)
  return pl.pallas_call(
      _finalize_body,
      out_shape=jax.ShapeDtypeStruct((S, D), jnp.float32),
  )(part_sum, part_sq)


# speedup vs baseline: 12.0082x; 1.4385x over previous
import functools

import jax
import jax.numpy as jnp
from jax import lax
from jax.experimental import pallas as pl
from jax.experimental.pallas import tpu as pltpu
from jax.experimental.pallas import tpu_sc as plsc

EPS = 1e-5
N = 100000
D = 256
S = 64
NC = 2
NS = 16
NW = NC * NS
B = 176
NG = D // 16

_HEX_Q, _HEX_R = divmod(N // 16, NW)
IDS_CHUNK = 16 * (_HEX_Q + 1)
IDS_PAD = 16 * _HEX_Q * NW + IDS_CHUNK
N_CHUNKS = -(-IDS_CHUNK // B)


def _sc_partials(feat, ids):
  mesh = plsc.VectorSubcoreMesh(core_axis_name="c", subcore_axis_name="s")

  @functools.partial(
      pl.kernel,
      mesh=mesh,
      out_type=[
          jax.ShapeDtypeStruct((NW, S, D), jnp.float32),
          jax.ShapeDtypeStruct((NW, S, D), jnp.float32),
      ],
      scratch_types=[
          pltpu.VMEM((IDS_CHUNK,), jnp.int32),
          pltpu.VMEM((2, B, D), jnp.float32),
          pltpu.VMEM((S, D), jnp.float32),
          pltpu.VMEM((S, D), jnp.float32),
          pltpu.VMEM((D,), jnp.float32),
          pltpu.VMEM((D,), jnp.float32),
          pltpu.SemaphoreType.DMA((2,)),
      ],
  )
  def k(feat_hbm, ids_hbm, out_sum, out_sq, ids_v, buf, acc_sum, acc_sq,
        car_sum, car_sq, sems):
    cid = lax.axis_index("c")
    sid = lax.axis_index("s")
    wid = sid * NC + cid
    lo = 16 * (_HEX_Q * wid + jnp.minimum(wid, _HEX_R))
    rpw = 16 * (_HEX_Q + jnp.where(wid < _HEX_R, 1, 0))
    hi = lo + rpw

    pltpu.sync_copy(ids_hbm.at[pl.ds(lo, IDS_CHUNK)], ids_v)

    z = jnp.zeros((16,), jnp.float32)
    for g in range(NG):
      car_sum[pl.ds(g * 16, 16)] = z
      car_sq[pl.ds(g * 16, 16)] = z

    def zero_row(i, carry):
      zz = jnp.zeros((16,), jnp.float32)
      for g in range(NG):
        acc_sum[i, pl.ds(g * 16, 16)] = zz
        acc_sq[i, pl.ds(g * 16, 16)] = zz
      return carry

    lax.fori_loop(0, S, zero_row, 0)

    cur0 = ids_v[pl.ds(0, 16)][0]

    def chunk_base(ci):
      return jnp.minimum(lo + ci * B + B, hi) - B

    def start_copy(ci, par):
      pltpu.async_copy(feat_hbm.at[pl.ds(chunk_base(ci), B), :], buf.at[par],
                       sems.at[par])

    start_copy(0, 0)

    def chunk_body(ci, cur):
      start = lo + ci * B
      end = jnp.minimum(start + B, hi)
      base = end - B
      par = lax.rem(ci, 2)
      pltpu.make_async_copy(feat_hbm.at[pl.ds(base, B), :], buf.at[par],
                            sems.at[par]).wait()
      nxt = jnp.minimum(ci + 1, N_CHUNKS - 1)
      start_copy(nxt, 1 - par)
      nsb = (end - start) // 16

      def sub_body(sb, cur):
        row0 = start + sb * 16
        segv = ids_v[pl.ds(row0 - lo, 16)]
        first = segv[0]
        last = segv[15]
        rl0 = row0 - base
        is_fast = jnp.logical_and(first == cur, first == last)
        n_slow = jnp.where(is_fast, 0, 1)

        def slow_body(_, c2):
          for g in range(NG):
            plsc.addupdate(acc_sum.at[cur, pl.ds(g * 16, 16)],
                           car_sum[pl.ds(g * 16, 16)])
            plsc.addupdate(acc_sq.at[cur, pl.ds(g * 16, 16)],
                           car_sq[pl.ds(g * 16, 16)])
            car_sum[pl.ds(g * 16, 16)] = jnp.zeros((16,), jnp.float32)
            car_sq[pl.ds(g * 16, 16)] = jnp.zeros((16,), jnp.float32)
          sv = ids_v[pl.ds(row0 - lo, 16)]
          for j in range(16):
            seg = sv[j]
            for g in range(NG):
              v = buf[par, rl0 + j, pl.ds(g * 16, 16)]
              plsc.addupdate(acc_sum.at[seg, pl.ds(g * 16, 16)], v)
              plsc.addupdate(acc_sq.at[seg, pl.ds(g * 16, 16)], v * v)
          return c2

        lax.fori_loop(0, n_slow, slow_body, 0)

        def fast_body(_, c2):
          for h in range(2):
            g0 = h * 8
            s_accs = [car_sum[pl.ds((g0 + g) * 16, 16)] for g in range(8)]
            q_accs = [car_sq[pl.ds((g0 + g) * 16, 16)] for g in range(8)]
            for j in range(16):
              row = buf.at[par, rl0 + j]
              for g in range(8):
                v = row[pl.ds((g0 + g) * 16, 16)]
                s_accs[g] = s_accs[g] + v
                q_accs[g] = q_accs[g] + v * v
            for g in range(8):
              car_sum[pl.ds((g0 + g) * 16, 16)] = s_accs[g]
              car_sq[pl.ds((g0 + g) * 16, 16)] = q_accs[g]
          return c2

        lax.fori_loop(0, 1 - n_slow, fast_body, 0)

        return last

      return lax.fori_loop(0, nsb, sub_body, cur)

    cur = lax.fori_loop(0, N_CHUNKS, chunk_body, cur0)

    pltpu.make_async_copy(
        feat_hbm.at[pl.ds(chunk_base(N_CHUNKS - 1), B), :], buf.at[0],
        sems.at[0]).wait()

    for g in range(NG):
      plsc.addupdate(acc_sum.at[cur, pl.ds(g * 16, 16)],
                     car_sum[pl.ds(g * 16, 16)])
      plsc.addupdate(acc_sq.at[cur, pl.ds(g * 16, 16)],
                     car_sq[pl.ds(g * 16, 16)])

    pltpu.sync_copy(acc_sum, out_sum.at[wid])
    pltpu.sync_copy(acc_sq, out_sq.at[wid])

  return k(feat, ids)


def _finalize_body(sum_ref, sq_ref, out_ref):
  s = jnp.sum(sum_ref[...], axis=0)
  q = jnp.sum(sq_ref[...], axis=0)
  out_ref[...] = jnp.sqrt(jax.nn.relu(q - s * s) + EPS)


@jax.jit
def kernel(feat, segment_ids):
  ids = segment_ids.astype(jnp.int32)
  ids = jnp.pad(ids, (0, IDS_PAD - N), constant_values=0)
  part_sum, part_sq = _sc_partials(feat, ids)
  return pl.pallas_call(
      _finalize_body,
      out_shape=jax.ShapeDtypeStruct((S, D), jnp.float32),
  )(part_sum, part_sq)
